# baseline (speedup 1.0000x reference)
import math

import jax
import jax.numpy as jnp
from jax.experimental import pallas as pl


def _sincos_body(o_ref):
    block, d = o_ref.shape
    i = pl.program_id(0)
    pos = (
        jax.lax.broadcasted_iota(jnp.int32, (block, d), 0) + i * block
    ).astype(jnp.float32)
    col = jax.lax.broadcasted_iota(jnp.int32, (block, d), 1)
    pair = (col >> 1).astype(jnp.float32)
    rate = jnp.exp(pair * jnp.float32(-2.0 * math.log(10000.0) / d))
    angle = pos * rate
    even = (col & 1) == 0
    o_ref[...] = jnp.where(even, jnp.sin(angle), jnp.cos(angle))


def kernel(x, pos_emb):
    seq = x.shape[1]
    d = pos_emb.shape[1]
    block = 512
    out = pl.pallas_call(
        _sincos_body,
        grid=(seq // block,),
        out_specs=pl.BlockSpec((block, d), lambda i: (i, 0)),
        out_shape=jax.ShapeDtypeStruct((seq, d), jnp.float32),
    )()
    return out
